# Initial kernel scaffold; baseline (speedup 1.0000x reference)
import jax
import jax.numpy as jnp
from jax.experimental import pallas as pl


def kernel(x, edge_index, edge_attr, We1, be1, We2, be2, W1, b1, W2, b2, g1, bt1, g2, bt2):
    raise NotImplementedError("write your pallas kernel here")


# speedup vs baseline: 5.5302x; 5.5302x over previous
import functools

import jax
import jax.numpy as jnp
from jax import lax
from jax.experimental import pallas as pl
from jax.experimental.pallas import tpu as pltpu
from jax.experimental.pallas import tpu_sc as plsc

_N = 10000
_E = 320000
_D_IN = 128
_D_HID = 256
_D_OUT = 128

_NC = 2
_NS = 16
_NW = _NC * _NS
_CHUNK = 128
_CPT = 79
_EPAD = _NW * _CPT * _CHUNK
_RPT = 624
_RLAST = _N - 15 * _RPT

_mesh = plsc.VectorSubcoreMesh(core_axis_name="c", subcore_axis_name="s")


def _ew_body(ea_ref, we1_ref, be1_ref, we2_ref, be2_ref, out_ref):
    i = pl.program_id(0)
    a = ea_ref[...]
    h = jnp.dot(a, we1_ref[...], preferred_element_type=jnp.float32)
    h = jnp.maximum(h + be1_ref[...], 0.0)
    ew = jnp.sum(h * we2_ref[...], axis=-1, keepdims=True) + be2_ref[...]
    ew = jnp.maximum(ew, 0.0)
    rows = lax.broadcasted_iota(jnp.int32, ew.shape, 0) + i * ew.shape[0]
    out_ref[...] = jnp.where(rows < _E, ew, 0.0)


def _ew_call(ea_p, We1, be1r, We2r, be2r):
    blk = 4096
    grid = _EPAD // blk
    return pl.pallas_call(
        _ew_body,
        grid=(grid,),
        in_specs=[
            pl.BlockSpec((blk, 16), lambda i: (i, 0)),
            pl.BlockSpec((16, 16), lambda i: (0, 0)),
            pl.BlockSpec((1, 16), lambda i: (0, 0)),
            pl.BlockSpec((1, 16), lambda i: (0, 0)),
            pl.BlockSpec((1, 1), lambda i: (0, 0)),
        ],
        out_specs=pl.BlockSpec((blk, 1), lambda i: (i, 0)),
        out_shape=jax.ShapeDtypeStruct((_EPAD, 1), jnp.float32),
    )(ea_p, We1, be1r, We2r, be2r)


@functools.partial(
    pl.kernel,
    out_type=jax.ShapeDtypeStruct((_NC * _N,), jnp.float32),
    mesh=_mesh,
    scratch_types=[
        pltpu.VMEM((_N,), jnp.float32),
        pltpu.VMEM((_CHUNK,), jnp.int32),
        pltpu.VMEM((_CHUNK,), jnp.float32),
        pltpu.VMEM_SHARED((_N,), jnp.float32),
        pltpu.SemaphoreType.DMA,
    ],
)
def _deg_kernel(dst_hbm, ew_hbm, out_hbm, zbuf, idx_v, ew_v, acc, sem):
    c = lax.axis_index("c")
    s = lax.axis_index("s")

    @pl.when(s == 0)
    def _zero():
        def zb(i, carry):
            zbuf[pl.ds(i * 16, 16)] = jnp.zeros((16,), jnp.float32)
            return carry
        lax.fori_loop(0, _N // 16, zb, 0)
        pltpu.sync_copy(zbuf, acc)

    plsc.subcore_barrier()

    base = (c * _NS + s) * _CPT

    def chunk(k, carry):
        off = (base + k) * _CHUNK
        pltpu.sync_copy(dst_hbm.at[pl.ds(off, _CHUNK)], idx_v)
        pltpu.sync_copy(ew_hbm.at[pl.ds(off, _CHUNK)], ew_v)
        pltpu.sync_copy(ew_v, acc.at[idx_v], add=True)
        return carry

    lax.fori_loop(0, _CPT, chunk, 0)
    plsc.subcore_barrier()

    @pl.when(s == 0)
    def _flush():
        pltpu.sync_copy(acc, zbuf)
        pltpu.sync_copy(zbuf, out_hbm.at[pl.ds(c * _N, _N)])


def _make_scatter(nh):
    @functools.partial(
        pl.kernel,
        out_type=jax.ShapeDtypeStruct((_NC, nh, _N, 128), jnp.float32),
        mesh=_mesh,
        compiler_params=pltpu.CompilerParams(needs_layout_passes=False),
        scratch_types=[
            pltpu.VMEM((_CHUNK,), jnp.int32),
            pltpu.VMEM((_CHUNK,), jnp.int32),
            pltpu.VMEM((_CHUNK,), jnp.float32),
            pltpu.VMEM((_CHUNK, 128), jnp.float32),
            pltpu.VMEM_SHARED((_N, 128), jnp.float32),
            pltpu.SemaphoreType.DMA,
        ],
    )
    def _k(*args):
        h_hbms = args[:nh]
        (src_hbm, dst_hbm, ew_hbm, out_hbm,
         src_v, dst_v, ew_v, rows_v, acc, sem) = args[nh:]
        c = lax.axis_index("c")
        s = lax.axis_index("s")
        tid = c * _NS + s

        for hh in range(nh):
            def zrow(i, carry):
                for j in range(8):
                    rows_v[i, pl.ds(j * 16, 16)] = jnp.zeros((16,), jnp.float32)
                return carry
            lax.fori_loop(0, _CHUNK, zrow, 0)
            r0 = s * _RPT

            @pl.when(s < _NS - 1)
            def _zero_main():
                for p in range(4):
                    pltpu.sync_copy(rows_v, acc.at[pl.ds(r0 + p * 128, 128)])
                pltpu.sync_copy(rows_v.at[pl.ds(0, _RPT - 512)],
                                acc.at[pl.ds(r0 + 512, _RPT - 512)])

            @pl.when(s == _NS - 1)
            def _zero_last():
                for p in range(5):
                    pltpu.sync_copy(
                        rows_v, acc.at[pl.ds(15 * _RPT + p * 128, 128)])

            plsc.subcore_barrier()

            def chunk(k, carry):
                off = (tid * _CPT + k) * _CHUNK
                pltpu.sync_copy(src_hbm.at[pl.ds(off, _CHUNK)], src_v)
                pltpu.sync_copy(dst_hbm.at[pl.ds(off, _CHUNK)], dst_v)
                pltpu.sync_copy(ew_hbm.at[pl.ds(off, _CHUNK)], ew_v)
                pltpu.async_copy(h_hbms[hh].at[src_v], rows_v, sem).wait()

                def edge(e, carry2):
                    bc = plsc.load_gather(
                        ew_v, [jnp.full((16,), e, jnp.int32)])
                    for j in range(8):
                        sl = pl.ds(j * 16, 16)
                        rows_v[e, sl] = rows_v[e, sl] * bc
                    return carry2

                lax.fori_loop(0, _CHUNK, edge, 0)
                pltpu.sync_copy(rows_v, acc.at[dst_v], add=True)
                return carry

            lax.fori_loop(0, _CPT, chunk, 0)
            plsc.subcore_barrier()

            @pl.when(s < _NS - 1)
            def _flush_main():
                pltpu.sync_copy(acc.at[pl.ds(r0, _RPT)],
                                out_hbm.at[c, hh, pl.ds(r0, _RPT)])

            @pl.when(s == _NS - 1)
            def _flush_last():
                pltpu.sync_copy(acc.at[pl.ds(15 * _RPT, _RLAST)],
                                out_hbm.at[c, hh, pl.ds(15 * _RPT, _RLAST)])

            plsc.subcore_barrier()

    return _k


_scatter2 = _make_scatter(2)
_scatter1 = _make_scatter(1)


def _h1_body(x_ref, w1_ref, degp_ref, ha_ref, hb_ref):
    h = jnp.dot(x_ref[...], w1_ref[...], preferred_element_type=jnp.float32)
    deg = degp_ref[0] + degp_ref[1] + 1.0
    dinv = lax.rsqrt(deg)
    hp = h * dinv
    ha_ref[...] = hp[:, :128]
    hb_ref[...] = hp[:, 128:]


def _h1_call(x, W1, degp3):
    blk = 2000
    grid = _N // blk
    return pl.pallas_call(
        _h1_body,
        grid=(grid,),
        in_specs=[
            pl.BlockSpec((blk, _D_IN), lambda i: (i, 0)),
            pl.BlockSpec((_D_IN, _D_HID), lambda i: (0, 0)),
            pl.BlockSpec((2, blk, 1), lambda i: (0, i, 0)),
        ],
        out_specs=[
            pl.BlockSpec((blk, 128), lambda i: (i, 0)),
            pl.BlockSpec((blk, 128), lambda i: (i, 0)),
        ],
        out_shape=[
            jax.ShapeDtypeStruct((_N, 128), jnp.float32),
            jax.ShapeDtypeStruct((_N, 128), jnp.float32),
        ],
    )(x, W1, degp3)


def _ep1_body(s1_ref, ha_ref, hb_ref, degp_ref, b1_ref, g1_ref, bt1_ref,
              w2_ref, out_ref):
    deg = degp_ref[0] + degp_ref[1] + 1.0
    dinv = lax.rsqrt(deg)
    sa = s1_ref[0, 0] + s1_ref[1, 0] + ha_ref[...]
    sb = s1_ref[0, 1] + s1_ref[1, 1] + hb_ref[...]
    t = dinv * jnp.concatenate([sa, sb], axis=-1) + b1_ref[...]
    mu = jnp.mean(t, axis=-1, keepdims=True)
    var = jnp.mean((t - mu) ** 2, axis=-1, keepdims=True)
    t = (t - mu) * lax.rsqrt(var + 1e-5) * g1_ref[...] + bt1_ref[...]
    t = jnp.maximum(t, 0.0)
    h2 = jnp.dot(t, w2_ref[...], preferred_element_type=jnp.float32)
    out_ref[...] = h2 * dinv


def _ep1_call(s1, ha, hb, degp3, b1r, g1r, bt1r, W2):
    blk = 2000
    grid = _N // blk
    return pl.pallas_call(
        _ep1_body,
        grid=(grid,),
        in_specs=[
            pl.BlockSpec((2, 2, blk, 128), lambda i: (0, 0, i, 0)),
            pl.BlockSpec((blk, 128), lambda i: (i, 0)),
            pl.BlockSpec((blk, 128), lambda i: (i, 0)),
            pl.BlockSpec((2, blk, 1), lambda i: (0, i, 0)),
            pl.BlockSpec((1, _D_HID), lambda i: (0, 0)),
            pl.BlockSpec((1, _D_HID), lambda i: (0, 0)),
            pl.BlockSpec((1, _D_HID), lambda i: (0, 0)),
            pl.BlockSpec((_D_HID, _D_OUT), lambda i: (0, 0)),
        ],
        out_specs=pl.BlockSpec((blk, 128), lambda i: (i, 0)),
        out_shape=jax.ShapeDtypeStruct((_N, 128), jnp.float32),
    )(s1, ha, hb, degp3, b1r, g1r, bt1r, W2)


def _ep2_body(s2_ref, h2p_ref, degp_ref, b2_ref, g2_ref, bt2_ref, out_ref):
    deg = degp_ref[0] + degp_ref[1] + 1.0
    dinv = lax.rsqrt(deg)
    t = dinv * (s2_ref[0, 0] + s2_ref[1, 0] + h2p_ref[...]) + b2_ref[...]
    mu = jnp.mean(t, axis=-1, keepdims=True)
    var = jnp.mean((t - mu) ** 2, axis=-1, keepdims=True)
    out_ref[...] = (t - mu) * lax.rsqrt(var + 1e-5) * g2_ref[...] + bt2_ref[...]


def _ep2_call(s2, h2p, degp3, b2r, g2r, bt2r):
    blk = 2000
    grid = _N // blk
    return pl.pallas_call(
        _ep2_body,
        grid=(grid,),
        in_specs=[
            pl.BlockSpec((2, 1, blk, 128), lambda i: (0, 0, i, 0)),
            pl.BlockSpec((blk, 128), lambda i: (i, 0)),
            pl.BlockSpec((2, blk, 1), lambda i: (0, i, 0)),
            pl.BlockSpec((1, _D_OUT), lambda i: (0, 0)),
            pl.BlockSpec((1, _D_OUT), lambda i: (0, 0)),
            pl.BlockSpec((1, _D_OUT), lambda i: (0, 0)),
        ],
        out_specs=pl.BlockSpec((blk, 128), lambda i: (i, 0)),
        out_shape=jax.ShapeDtypeStruct((_N, _D_OUT), jnp.float32),
    )(s2, h2p, degp3, b2r, g2r, bt2r)


def kernel(x, edge_index, edge_attr, We1, be1, We2, be2,
           W1, b1, W2, b2, g1, bt1, g2, bt2):
    src = edge_index[0].astype(jnp.int32)
    dst = edge_index[1].astype(jnp.int32)
    pad = _EPAD - _E
    ea_p = jnp.pad(edge_attr, ((0, pad), (0, 0)))
    src_p = jnp.pad(src, (0, pad))
    dst_p = jnp.pad(dst, (0, pad))

    ew_p = _ew_call(ea_p, We1, be1.reshape(1, 16), We2.reshape(1, 16),
                    be2.reshape(1, 1)).reshape(_EPAD)
    degp = _deg_kernel(dst_p, ew_p)
    degp3 = degp.reshape(_NC, _N, 1)
    ha, hb = _h1_call(x, W1, degp3)
    s1 = _scatter2(ha, hb, src_p, dst_p, ew_p)
    h2p = _ep1_call(s1, ha, hb, degp3, b1.reshape(1, _D_HID),
                    g1.reshape(1, _D_HID), bt1.reshape(1, _D_HID), W2)
    s2 = _scatter1(h2p, src_p, dst_p, ew_p)
    out = _ep2_call(s2, h2p, degp3, b2.reshape(1, _D_OUT),
                    g2.reshape(1, _D_OUT), bt2.reshape(1, _D_OUT))
    return out


# speedup vs baseline: 5.9124x; 1.0691x over previous
import functools

import jax
import jax.numpy as jnp
from jax import lax
from jax.experimental import pallas as pl
from jax.experimental.pallas import tpu as pltpu
from jax.experimental.pallas import tpu_sc as plsc

_N = 10000
_E = 320000
_D_IN = 128
_D_HID = 256
_D_OUT = 128

_NC = 2
_NS = 16
_NW = _NC * _NS
_CHUNK = 128
_CPT = 80
_STG = 2
_CPS = _CPT // _STG
_EPAD = _NW * _CPT * _CHUNK
_RPT = 624
_RLAST = _N - 15 * _RPT

_mesh = plsc.VectorSubcoreMesh(core_axis_name="c", subcore_axis_name="s")


def _ew_body(ea_ref, we1_ref, be1_ref, we2_ref, be2_ref, out_ref):
    i = pl.program_id(0)
    a = ea_ref[...]
    h = jnp.dot(a, we1_ref[...], preferred_element_type=jnp.float32)
    h = jnp.maximum(h + be1_ref[...], 0.0)
    ew = jnp.sum(h * we2_ref[...], axis=-1, keepdims=True) + be2_ref[...]
    ew = jnp.maximum(ew, 0.0)
    rows = lax.broadcasted_iota(jnp.int32, ew.shape, 0) + i * ew.shape[0]
    out_ref[...] = jnp.where(rows < _E, ew, 0.0)


def _ew_call(ea_p, We1, be1r, We2r, be2r):
    blk = 4096
    grid = _EPAD // blk
    return pl.pallas_call(
        _ew_body,
        grid=(grid,),
        in_specs=[
            pl.BlockSpec((blk, 16), lambda i: (i, 0)),
            pl.BlockSpec((16, 16), lambda i: (0, 0)),
            pl.BlockSpec((1, 16), lambda i: (0, 0)),
            pl.BlockSpec((1, 16), lambda i: (0, 0)),
            pl.BlockSpec((1, 1), lambda i: (0, 0)),
        ],
        out_specs=pl.BlockSpec((blk, 1), lambda i: (i, 0)),
        out_shape=jax.ShapeDtypeStruct((_EPAD, 1), jnp.float32),
    )(ea_p, We1, be1r, We2r, be2r)


@functools.partial(
    pl.kernel,
    out_type=jax.ShapeDtypeStruct((_NC * _N,), jnp.float32),
    mesh=_mesh,
    compiler_params=pltpu.CompilerParams(needs_layout_passes=False),
    scratch_types=[
        pltpu.VMEM((_N,), jnp.float32),
        pltpu.VMEM((_CPT, _CHUNK), jnp.int32),
        pltpu.VMEM((_CPT, _CHUNK), jnp.float32),
        pltpu.VMEM_SHARED((_N,), jnp.float32),
        pltpu.SemaphoreType.DMA,
        pltpu.SemaphoreType.DMA,
    ],
)
def _deg_kernel(dst_hbm, ew_hbm, out_hbm, zbuf, dst2d, ew2d, acc, lsem, ssem):
    c = lax.axis_index("c")
    s = lax.axis_index("s")
    tid = c * _NS + s

    @pl.when(s == 0)
    def _zero():
        def zb(i, carry):
            zbuf[pl.ds(i * 16, 16)] = jnp.zeros((16,), jnp.float32)
            return carry
        lax.fori_loop(0, _N // 16, zb, 0)
        pltpu.sync_copy(zbuf, acc)

    pltpu.async_copy(dst_hbm.at[pl.ds(tid * _CPT, _CPT)], dst2d, lsem)
    pltpu.async_copy(ew_hbm.at[pl.ds(tid * _CPT, _CPT)], ew2d, lsem)
    pltpu.make_async_copy(dst_hbm.at[pl.ds(0, _CPT)], dst2d, lsem).wait()
    pltpu.make_async_copy(ew_hbm.at[pl.ds(0, _CPT)], ew2d, lsem).wait()
    plsc.subcore_barrier()

    def chunk(k, carry):
        pltpu.async_copy(ew2d.at[k], acc.at[dst2d.at[k]], ssem, add=True)
        return carry

    lax.fori_loop(0, _CPT, chunk, 0)

    def drain(k, carry):
        pltpu.make_async_copy(ew2d.at[0], acc.at[dst2d.at[0]], ssem).wait()
        return carry

    lax.fori_loop(0, _CPT, drain, 0)
    plsc.subcore_barrier()

    @pl.when(s == 0)
    def _flush():
        pltpu.sync_copy(acc, zbuf)
        pltpu.sync_copy(zbuf, out_hbm.at[pl.ds(c * _N, _N)])


def _make_scatter(nh):
    @functools.partial(
        pl.kernel,
        out_type=jax.ShapeDtypeStruct((_NC, nh, _N, 128), jnp.float32),
        mesh=_mesh,
        compiler_params=pltpu.CompilerParams(needs_layout_passes=False),
        scratch_types=[
            pltpu.VMEM((_CPS, _CHUNK), jnp.int32),
            pltpu.VMEM((_CPS, _CHUNK), jnp.int32),
            pltpu.VMEM((_CPS, _CHUNK), jnp.float32),
            pltpu.VMEM((_CHUNK, 128), jnp.float32),
            pltpu.VMEM((_CHUNK, 128), jnp.float32),
            pltpu.VMEM_SHARED((_N, 128), jnp.float32),
            pltpu.SemaphoreType.DMA,
            pltpu.SemaphoreType.DMA,
            pltpu.SemaphoreType.DMA,
        ],
    )
    def _k(*args):
        h_hbms = args[:nh]
        (src_hbm, dst_hbm, ew_hbm, out_hbm,
         src2d, dst2d, ew2d, rows_a, rows_b, acc, lsem, gsa, gsb) = args[nh:]
        c = lax.axis_index("c")
        s = lax.axis_index("s")
        tid = c * _NS + s

        def scale(rows_v, k):
            def edge(e, carry2):
                bc = plsc.load_gather(
                    ew2d, [jnp.full((16,), k, jnp.int32),
                           jnp.full((16,), e, jnp.int32)])
                for j in range(8):
                    sl = pl.ds(j * 16, 16)
                    rows_v[e, sl] = rows_v[e, sl] * bc
                return carry2

            lax.fori_loop(0, _CHUNK, edge, 0, unroll=2)

        for hh in range(nh):
            h_hbm = h_hbms[hh]
            def zrow(i, carry):
                for j in range(8):
                    rows_a[i, pl.ds(j * 16, 16)] = jnp.zeros(
                        (16,), jnp.float32)
                return carry
            lax.fori_loop(0, _CHUNK, zrow, 0)
            r0 = s * _RPT

            @pl.when(s < _NS - 1)
            def _zero_main():
                for p in range(4):
                    pltpu.sync_copy(rows_a, acc.at[pl.ds(r0 + p * 128, 128)])
                pltpu.sync_copy(rows_a.at[pl.ds(0, _RPT - 512)],
                                acc.at[pl.ds(r0 + 512, _RPT - 512)])

            @pl.when(s == _NS - 1)
            def _zero_last():
                for p in range(5):
                    pltpu.sync_copy(
                        rows_a, acc.at[pl.ds(15 * _RPT + p * 128, 128)])

            plsc.subcore_barrier()

            for st in range(_STG):
                b0 = tid * _CPT + st * _CPS
                pltpu.async_copy(src_hbm.at[pl.ds(b0, _CPS)], src2d, lsem)
                pltpu.async_copy(dst_hbm.at[pl.ds(b0, _CPS)], dst2d, lsem)
                pltpu.async_copy(ew_hbm.at[pl.ds(b0, _CPS)], ew2d, lsem)
                pltpu.make_async_copy(
                    src_hbm.at[pl.ds(0, _CPS)], src2d, lsem).wait()
                pltpu.make_async_copy(
                    dst_hbm.at[pl.ds(0, _CPS)], dst2d, lsem).wait()
                pltpu.make_async_copy(
                    ew_hbm.at[pl.ds(0, _CPS)], ew2d, lsem).wait()

                pltpu.async_copy(h_hbm.at[src2d.at[0]], rows_a, gsa)
                pltpu.async_copy(h_hbm.at[src2d.at[1]], rows_b, gsb)

                def pipe(g, carry):
                    k0 = 2 * g
                    pltpu.make_async_copy(
                        h_hbm.at[src2d.at[0]], rows_a, gsa).wait()
                    scale(rows_a, k0)
                    pltpu.sync_copy(rows_a, acc.at[dst2d.at[k0]], add=True)

                    @pl.when(k0 + 2 < _CPS)
                    def _pf_a():
                        pltpu.async_copy(
                            h_hbm.at[src2d.at[k0 + 2]], rows_a, gsa)

                    pltpu.make_async_copy(
                        h_hbm.at[src2d.at[0]], rows_b, gsb).wait()
                    scale(rows_b, k0 + 1)
                    pltpu.sync_copy(rows_b, acc.at[dst2d.at[k0 + 1]],
                                    add=True)

                    @pl.when(k0 + 3 < _CPS)
                    def _pf_b():
                        pltpu.async_copy(
                            h_hbm.at[src2d.at[k0 + 3]], rows_b, gsb)

                    return carry

                lax.fori_loop(0, _CPS // 2, pipe, 0)

            plsc.subcore_barrier()

            @pl.when(s < _NS - 1)
            def _flush_main():
                pltpu.sync_copy(acc.at[pl.ds(r0, _RPT)],
                                out_hbm.at[c, hh, pl.ds(r0, _RPT)])

            @pl.when(s == _NS - 1)
            def _flush_last():
                pltpu.sync_copy(acc.at[pl.ds(15 * _RPT, _RLAST)],
                                out_hbm.at[c, hh, pl.ds(15 * _RPT, _RLAST)])

            plsc.subcore_barrier()

    return _k


_scatter2 = _make_scatter(2)
_scatter1 = _make_scatter(1)


def _h1_body(x_ref, w1_ref, degp_ref, ha_ref, hb_ref):
    h = jnp.dot(x_ref[...], w1_ref[...], preferred_element_type=jnp.float32)
    deg = degp_ref[0] + degp_ref[1] + 1.0
    dinv = lax.rsqrt(deg)
    hp = h * dinv
    ha_ref[...] = hp[:, :128]
    hb_ref[...] = hp[:, 128:]


def _h1_call(x, W1, degp3):
    blk = 2000
    grid = _N // blk
    return pl.pallas_call(
        _h1_body,
        grid=(grid,),
        in_specs=[
            pl.BlockSpec((blk, _D_IN), lambda i: (i, 0)),
            pl.BlockSpec((_D_IN, _D_HID), lambda i: (0, 0)),
            pl.BlockSpec((2, blk, 1), lambda i: (0, i, 0)),
        ],
        out_specs=[
            pl.BlockSpec((blk, 128), lambda i: (i, 0)),
            pl.BlockSpec((blk, 128), lambda i: (i, 0)),
        ],
        out_shape=[
            jax.ShapeDtypeStruct((_N, 128), jnp.float32),
            jax.ShapeDtypeStruct((_N, 128), jnp.float32),
        ],
    )(x, W1, degp3)


def _ep1_body(s1_ref, ha_ref, hb_ref, degp_ref, b1_ref, g1_ref, bt1_ref,
              w2_ref, out_ref):
    deg = degp_ref[0] + degp_ref[1] + 1.0
    dinv = lax.rsqrt(deg)
    sa = s1_ref[0, 0] + s1_ref[1, 0] + ha_ref[...]
    sb = s1_ref[0, 1] + s1_ref[1, 1] + hb_ref[...]
    t = dinv * jnp.concatenate([sa, sb], axis=-1) + b1_ref[...]
    mu = jnp.mean(t, axis=-1, keepdims=True)
    var = jnp.mean((t - mu) ** 2, axis=-1, keepdims=True)
    t = (t - mu) * lax.rsqrt(var + 1e-5) * g1_ref[...] + bt1_ref[...]
    t = jnp.maximum(t, 0.0)
    h2 = jnp.dot(t, w2_ref[...], preferred_element_type=jnp.float32)
    out_ref[...] = h2 * dinv


def _ep1_call(s1, ha, hb, degp3, b1r, g1r, bt1r, W2):
    blk = 2000
    grid = _N // blk
    return pl.pallas_call(
        _ep1_body,
        grid=(grid,),
        in_specs=[
            pl.BlockSpec((2, 2, blk, 128), lambda i: (0, 0, i, 0)),
            pl.BlockSpec((blk, 128), lambda i: (i, 0)),
            pl.BlockSpec((blk, 128), lambda i: (i, 0)),
            pl.BlockSpec((2, blk, 1), lambda i: (0, i, 0)),
            pl.BlockSpec((1, _D_HID), lambda i: (0, 0)),
            pl.BlockSpec((1, _D_HID), lambda i: (0, 0)),
            pl.BlockSpec((1, _D_HID), lambda i: (0, 0)),
            pl.BlockSpec((_D_HID, _D_OUT), lambda i: (0, 0)),
        ],
        out_specs=pl.BlockSpec((blk, 128), lambda i: (i, 0)),
        out_shape=jax.ShapeDtypeStruct((_N, 128), jnp.float32),
    )(s1, ha, hb, degp3, b1r, g1r, bt1r, W2)


def _ep2_body(s2_ref, h2p_ref, degp_ref, b2_ref, g2_ref, bt2_ref, out_ref):
    deg = degp_ref[0] + degp_ref[1] + 1.0
    dinv = lax.rsqrt(deg)
    t = dinv * (s2_ref[0, 0] + s2_ref[1, 0] + h2p_ref[...]) + b2_ref[...]
    mu = jnp.mean(t, axis=-1, keepdims=True)
    var = jnp.mean((t - mu) ** 2, axis=-1, keepdims=True)
    out_ref[...] = (t - mu) * lax.rsqrt(var + 1e-5) * g2_ref[...] + bt2_ref[...]


def _ep2_call(s2, h2p, degp3, b2r, g2r, bt2r):
    blk = 2000
    grid = _N // blk
    return pl.pallas_call(
        _ep2_body,
        grid=(grid,),
        in_specs=[
            pl.BlockSpec((2, 1, blk, 128), lambda i: (0, 0, i, 0)),
            pl.BlockSpec((blk, 128), lambda i: (i, 0)),
            pl.BlockSpec((2, blk, 1), lambda i: (0, i, 0)),
            pl.BlockSpec((1, _D_OUT), lambda i: (0, 0)),
            pl.BlockSpec((1, _D_OUT), lambda i: (0, 0)),
            pl.BlockSpec((1, _D_OUT), lambda i: (0, 0)),
        ],
        out_specs=pl.BlockSpec((blk, 128), lambda i: (i, 0)),
        out_shape=jax.ShapeDtypeStruct((_N, _D_OUT), jnp.float32),
    )(s2, h2p, degp3, b2r, g2r, bt2r)


def kernel(x, edge_index, edge_attr, We1, be1, We2, be2,
           W1, b1, W2, b2, g1, bt1, g2, bt2):
    src = edge_index[0].astype(jnp.int32)
    dst = edge_index[1].astype(jnp.int32)
    pad = _EPAD - _E
    ea_p = jnp.pad(edge_attr, ((0, pad), (0, 0)))
    src_p = jnp.pad(src, (0, pad))
    dst_p = jnp.pad(dst, (0, pad))

    nrow = _NW * _CPT
    ew2 = _ew_call(ea_p, We1, be1.reshape(1, 16), We2.reshape(1, 16),
                   be2.reshape(1, 1)).reshape(nrow, _CHUNK)
    src2 = src_p.reshape(nrow, _CHUNK)
    dst2 = dst_p.reshape(nrow, _CHUNK)
    degp = _deg_kernel(dst2, ew2)
    degp3 = degp.reshape(_NC, _N, 1)
    ha, hb = _h1_call(x, W1, degp3)
    s1 = _scatter2(ha, hb, src2, dst2, ew2)
    h2p = _ep1_call(s1, ha, hb, degp3, b1.reshape(1, _D_HID),
                    g1.reshape(1, _D_HID), bt1.reshape(1, _D_HID), W2)
    s2 = _scatter1(h2p, src2, dst2, ew2)
    out = _ep2_call(s2, h2p, degp3, b2.reshape(1, _D_OUT),
                    g2.reshape(1, _D_OUT), bt2.reshape(1, _D_OUT))
    return out


# speedup vs baseline: 6.7239x; 1.1373x over previous
import functools

import jax
import jax.numpy as jnp
from jax import lax
from jax.experimental import pallas as pl
from jax.experimental.pallas import tpu as pltpu
from jax.experimental.pallas import tpu_sc as plsc

_N = 10000
_E = 320000
_D_IN = 128
_D_HID = 256
_D_OUT = 128

_NC = 2
_NS = 16
_NW = _NC * _NS
_CHUNK = 128
_CPT = 80
_STG = 2
_CPS = _CPT // _STG
_EPAD = _NW * _CPT * _CHUNK
_RPT = 624
_RLAST = _N - 15 * _RPT

_mesh = plsc.VectorSubcoreMesh(core_axis_name="c", subcore_axis_name="s")


def _ew_body(ar_ref, w1b_ref, b1t_ref, w2b_ref, be2_ref, out_ref):
    h = jnp.dot(ar_ref[...], w1b_ref[...], preferred_element_type=jnp.float32)
    h = jnp.maximum(h + b1t_ref[...], 0.0)
    ew = jnp.dot(h, w2b_ref[...], preferred_element_type=jnp.float32)
    out_ref[...] = jnp.maximum(ew + be2_ref[...], 0.0)


def _ew_call(ar, w1b, b1t, w2b, be2r):
    blk = 4000
    grid = (_E // 8) // blk
    return pl.pallas_call(
        _ew_body,
        grid=(grid,),
        in_specs=[
            pl.BlockSpec((blk, 128), lambda i: (i, 0)),
            pl.BlockSpec((128, 128), lambda i: (0, 0)),
            pl.BlockSpec((1, 128), lambda i: (0, 0)),
            pl.BlockSpec((128, 8), lambda i: (0, 0)),
            pl.BlockSpec((1, 1), lambda i: (0, 0)),
        ],
        out_specs=pl.BlockSpec((blk, 8), lambda i: (i, 0)),
        out_shape=jax.ShapeDtypeStruct((_E // 8, 8), jnp.float32),
    )(ar, w1b, b1t, w2b, be2r)


@functools.partial(
    pl.kernel,
    out_type=jax.ShapeDtypeStruct((_NC * _N,), jnp.float32),
    mesh=_mesh,
    compiler_params=pltpu.CompilerParams(needs_layout_passes=False),
    scratch_types=[
        pltpu.VMEM((_N,), jnp.float32),
        pltpu.VMEM((_CPT, _CHUNK), jnp.int32),
        pltpu.VMEM((_CPT, _CHUNK), jnp.float32),
        pltpu.VMEM_SHARED((_N,), jnp.float32),
        pltpu.SemaphoreType.DMA,
        pltpu.SemaphoreType.DMA,
    ],
)
def _deg_kernel(dst_hbm, ew_hbm, out_hbm, zbuf, dst2d, ew2d, acc, lsem, ssem):
    c = lax.axis_index("c")
    s = lax.axis_index("s")
    tid = c * _NS + s

    @pl.when(s == 0)
    def _zero():
        def zb(i, carry):
            zbuf[pl.ds(i * 16, 16)] = jnp.zeros((16,), jnp.float32)
            return carry
        lax.fori_loop(0, _N // 16, zb, 0)
        pltpu.sync_copy(zbuf, acc)

    pltpu.async_copy(dst_hbm.at[pl.ds(tid * _CPT, _CPT)], dst2d, lsem)
    pltpu.async_copy(ew_hbm.at[pl.ds(tid * _CPT, _CPT)], ew2d, lsem)
    pltpu.make_async_copy(dst_hbm.at[pl.ds(0, _CPT)], dst2d, lsem).wait()
    pltpu.make_async_copy(ew_hbm.at[pl.ds(0, _CPT)], ew2d, lsem).wait()
    plsc.subcore_barrier()

    def chunk(k, carry):
        pltpu.async_copy(ew2d.at[k], acc.at[dst2d.at[k]], ssem, add=True)
        return carry

    lax.fori_loop(0, _CPT, chunk, 0)

    def drain(k, carry):
        pltpu.make_async_copy(ew2d.at[0], acc.at[dst2d.at[0]], ssem).wait()
        return carry

    lax.fori_loop(0, _CPT, drain, 0)
    plsc.subcore_barrier()

    @pl.when(s == 0)
    def _flush():
        pltpu.sync_copy(acc, zbuf)
        pltpu.sync_copy(zbuf, out_hbm.at[pl.ds(c * _N, _N)])


def _make_scatter(nh):
    @functools.partial(
        pl.kernel,
        out_type=jax.ShapeDtypeStruct((_NC, nh, _N, 128), jnp.float32),
        mesh=_mesh,
        compiler_params=pltpu.CompilerParams(needs_layout_passes=False),
        scratch_types=[
            pltpu.VMEM((_CPS, _CHUNK), jnp.int32),
            pltpu.VMEM((_CPS, _CHUNK), jnp.int32),
            pltpu.VMEM((_CPS, _CHUNK), jnp.float32),
            pltpu.VMEM((_CHUNK, 128), jnp.float32),
            pltpu.VMEM((_CHUNK, 128), jnp.float32),
            pltpu.VMEM_SHARED((_N, 128), jnp.float32),
            pltpu.SemaphoreType.DMA,
            pltpu.SemaphoreType.DMA,
            pltpu.SemaphoreType.DMA,
        ],
    )
    def _k(*args):
        h_hbms = args[:nh]
        (src_hbm, dst_hbm, ew_hbm, out_hbm,
         src2d, dst2d, ew2d, rows_a, rows_b, acc, lsem, gsa, gsb) = args[nh:]
        c = lax.axis_index("c")
        s = lax.axis_index("s")
        tid = c * _NS + s

        def scale(rows_v, k):
            def edge(e, carry2):
                bc = plsc.load_gather(
                    ew2d, [jnp.full((16,), k, jnp.int32),
                           jnp.full((16,), e, jnp.int32)])
                for j in range(8):
                    sl = pl.ds(j * 16, 16)
                    rows_v[e, sl] = rows_v[e, sl] * bc
                return carry2

            lax.fori_loop(0, _CHUNK, edge, 0, unroll=2)

        for hh in range(nh):
            h_hbm = h_hbms[hh]
            def zrow(i, carry):
                for j in range(8):
                    rows_a[i, pl.ds(j * 16, 16)] = jnp.zeros(
                        (16,), jnp.float32)
                return carry
            lax.fori_loop(0, _CHUNK, zrow, 0)
            r0 = s * _RPT

            @pl.when(s < _NS - 1)
            def _zero_main():
                for p in range(4):
                    pltpu.sync_copy(rows_a, acc.at[pl.ds(r0 + p * 128, 128)])
                pltpu.sync_copy(rows_a.at[pl.ds(0, _RPT - 512)],
                                acc.at[pl.ds(r0 + 512, _RPT - 512)])

            @pl.when(s == _NS - 1)
            def _zero_last():
                for p in range(5):
                    pltpu.sync_copy(
                        rows_a, acc.at[pl.ds(15 * _RPT + p * 128, 128)])

            plsc.subcore_barrier()

            for st in range(_STG):
                b0 = tid * _CPT + st * _CPS
                pltpu.async_copy(src_hbm.at[pl.ds(b0, _CPS)], src2d, lsem)
                pltpu.async_copy(dst_hbm.at[pl.ds(b0, _CPS)], dst2d, lsem)
                pltpu.async_copy(ew_hbm.at[pl.ds(b0, _CPS)], ew2d, lsem)
                pltpu.make_async_copy(
                    src_hbm.at[pl.ds(0, _CPS)], src2d, lsem).wait()
                pltpu.make_async_copy(
                    dst_hbm.at[pl.ds(0, _CPS)], dst2d, lsem).wait()
                pltpu.make_async_copy(
                    ew_hbm.at[pl.ds(0, _CPS)], ew2d, lsem).wait()

                pltpu.async_copy(h_hbm.at[src2d.at[0]], rows_a, gsa)
                pltpu.async_copy(h_hbm.at[src2d.at[1]], rows_b, gsb)

                def pipe(g, carry):
                    k0 = 2 * g
                    pltpu.make_async_copy(
                        h_hbm.at[src2d.at[0]], rows_a, gsa).wait()
                    scale(rows_a, k0)
                    pltpu.sync_copy(rows_a, acc.at[dst2d.at[k0]], add=True)

                    @pl.when(k0 + 2 < _CPS)
                    def _pf_a():
                        pltpu.async_copy(
                            h_hbm.at[src2d.at[k0 + 2]], rows_a, gsa)

                    pltpu.make_async_copy(
                        h_hbm.at[src2d.at[0]], rows_b, gsb).wait()
                    scale(rows_b, k0 + 1)
                    pltpu.sync_copy(rows_b, acc.at[dst2d.at[k0 + 1]],
                                    add=True)

                    @pl.when(k0 + 3 < _CPS)
                    def _pf_b():
                        pltpu.async_copy(
                            h_hbm.at[src2d.at[k0 + 3]], rows_b, gsb)

                    return carry

                lax.fori_loop(0, _CPS // 2, pipe, 0)

            plsc.subcore_barrier()

            @pl.when(s < _NS - 1)
            def _flush_main():
                pltpu.sync_copy(acc.at[pl.ds(r0, _RPT)],
                                out_hbm.at[c, hh, pl.ds(r0, _RPT)])

            @pl.when(s == _NS - 1)
            def _flush_last():
                pltpu.sync_copy(acc.at[pl.ds(15 * _RPT, _RLAST)],
                                out_hbm.at[c, hh, pl.ds(15 * _RPT, _RLAST)])

            plsc.subcore_barrier()

    return _k


_scatter2 = _make_scatter(2)
_scatter1 = _make_scatter(1)


def _h1_body(x_ref, w1_ref, degp_ref, ha_ref, hb_ref):
    h = jnp.dot(x_ref[...], w1_ref[...], preferred_element_type=jnp.float32)
    deg = degp_ref[0] + degp_ref[1] + 1.0
    dinv = lax.rsqrt(deg)
    hp = h * dinv
    ha_ref[...] = hp[:, :128]
    hb_ref[...] = hp[:, 128:]


def _h1_call(x, W1, degp3):
    blk = 2000
    grid = _N // blk
    return pl.pallas_call(
        _h1_body,
        grid=(grid,),
        in_specs=[
            pl.BlockSpec((blk, _D_IN), lambda i: (i, 0)),
            pl.BlockSpec((_D_IN, _D_HID), lambda i: (0, 0)),
            pl.BlockSpec((2, blk, 1), lambda i: (0, i, 0)),
        ],
        out_specs=[
            pl.BlockSpec((blk, 128), lambda i: (i, 0)),
            pl.BlockSpec((blk, 128), lambda i: (i, 0)),
        ],
        out_shape=[
            jax.ShapeDtypeStruct((_N, 128), jnp.float32),
            jax.ShapeDtypeStruct((_N, 128), jnp.float32),
        ],
    )(x, W1, degp3)


def _ep1_body(s1_ref, ha_ref, hb_ref, degp_ref, b1_ref, g1_ref, bt1_ref,
              w2_ref, out_ref):
    deg = degp_ref[0] + degp_ref[1] + 1.0
    dinv = lax.rsqrt(deg)
    sa = s1_ref[0, 0] + s1_ref[1, 0] + ha_ref[...]
    sb = s1_ref[0, 1] + s1_ref[1, 1] + hb_ref[...]
    t = dinv * jnp.concatenate([sa, sb], axis=-1) + b1_ref[...]
    mu = jnp.mean(t, axis=-1, keepdims=True)
    var = jnp.mean((t - mu) ** 2, axis=-1, keepdims=True)
    t = (t - mu) * lax.rsqrt(var + 1e-5) * g1_ref[...] + bt1_ref[...]
    t = jnp.maximum(t, 0.0)
    h2 = jnp.dot(t, w2_ref[...], preferred_element_type=jnp.float32)
    out_ref[...] = h2 * dinv


def _ep1_call(s1, ha, hb, degp3, b1r, g1r, bt1r, W2):
    blk = 2000
    grid = _N // blk
    return pl.pallas_call(
        _ep1_body,
        grid=(grid,),
        in_specs=[
            pl.BlockSpec((2, 2, blk, 128), lambda i: (0, 0, i, 0)),
            pl.BlockSpec((blk, 128), lambda i: (i, 0)),
            pl.BlockSpec((blk, 128), lambda i: (i, 0)),
            pl.BlockSpec((2, blk, 1), lambda i: (0, i, 0)),
            pl.BlockSpec((1, _D_HID), lambda i: (0, 0)),
            pl.BlockSpec((1, _D_HID), lambda i: (0, 0)),
            pl.BlockSpec((1, _D_HID), lambda i: (0, 0)),
            pl.BlockSpec((_D_HID, _D_OUT), lambda i: (0, 0)),
        ],
        out_specs=pl.BlockSpec((blk, 128), lambda i: (i, 0)),
        out_shape=jax.ShapeDtypeStruct((_N, 128), jnp.float32),
    )(s1, ha, hb, degp3, b1r, g1r, bt1r, W2)


def _ep2_body(s2_ref, h2p_ref, degp_ref, b2_ref, g2_ref, bt2_ref, out_ref):
    deg = degp_ref[0] + degp_ref[1] + 1.0
    dinv = lax.rsqrt(deg)
    t = dinv * (s2_ref[0, 0] + s2_ref[1, 0] + h2p_ref[...]) + b2_ref[...]
    mu = jnp.mean(t, axis=-1, keepdims=True)
    var = jnp.mean((t - mu) ** 2, axis=-1, keepdims=True)
    out_ref[...] = (t - mu) * lax.rsqrt(var + 1e-5) * g2_ref[...] + bt2_ref[...]


def _ep2_call(s2, h2p, degp3, b2r, g2r, bt2r):
    blk = 2000
    grid = _N // blk
    return pl.pallas_call(
        _ep2_body,
        grid=(grid,),
        in_specs=[
            pl.BlockSpec((2, 1, blk, 128), lambda i: (0, 0, i, 0)),
            pl.BlockSpec((blk, 128), lambda i: (i, 0)),
            pl.BlockSpec((2, blk, 1), lambda i: (0, i, 0)),
            pl.BlockSpec((1, _D_OUT), lambda i: (0, 0)),
            pl.BlockSpec((1, _D_OUT), lambda i: (0, 0)),
            pl.BlockSpec((1, _D_OUT), lambda i: (0, 0)),
        ],
        out_specs=pl.BlockSpec((blk, 128), lambda i: (i, 0)),
        out_shape=jax.ShapeDtypeStruct((_N, _D_OUT), jnp.float32),
    )(s2, h2p, degp3, b2r, g2r, bt2r)


def kernel(x, edge_index, edge_attr, We1, be1, We2, be2,
           W1, b1, W2, b2, g1, bt1, g2, bt2):
    src = edge_index[0].astype(jnp.int32)
    dst = edge_index[1].astype(jnp.int32)
    pad = _EPAD - _E
    src_p = jnp.pad(src, (0, pad))
    dst_p = jnp.pad(dst, (0, pad))

    nrow = _NW * _CPT
    eye8 = jnp.eye(8, dtype=jnp.float32)
    ew = _ew_call(edge_attr.reshape(_E // 8, 128), jnp.kron(eye8, We1),
                  jnp.tile(be1, 8).reshape(1, 128), jnp.kron(eye8, We2),
                  be2.reshape(1, 1)).reshape(_E)
    ew2 = jnp.pad(ew, (0, pad)).reshape(nrow, _CHUNK)
    src2 = src_p.reshape(nrow, _CHUNK)
    dst2 = dst_p.reshape(nrow, _CHUNK)
    degp = _deg_kernel(dst2, ew2)
    degp3 = degp.reshape(_NC, _N, 1)
    ha, hb = _h1_call(x, W1, degp3)
    s1 = _scatter2(ha, hb, src2, dst2, ew2)
    h2p = _ep1_call(s1, ha, hb, degp3, b1.reshape(1, _D_HID),
                    g1.reshape(1, _D_HID), bt1.reshape(1, _D_HID), W2)
    s2 = _scatter1(h2p, src2, dst2, ew2)
    out = _ep2_call(s2, h2p, degp3, b2.reshape(1, _D_OUT),
                    g2.reshape(1, _D_OUT), bt2.reshape(1, _D_OUT))
    return out
